# baseline (speedup 1.0000x reference)
import functools

import jax
import jax.numpy as jnp
from jax import lax
from jax.experimental import pallas as pl
from jax.experimental.pallas import tpu as pltpu
from jax.experimental.pallas import tpu_sc as plsc

N = 10000
D = 128
E = 320000
NC = 2
NS = 16
NW = NC * NS
CHUNK = 128
NCH = 80
EPW = NCH * CHUNK
EPAD = NW * EPW
NPAD = 10112
STRIPE = NPAD // NS

_mesh = plsc.VectorSubcoreMesh(core_axis_name="c", subcore_axis_name="s")


def _deg_body(dst3, out, didx, hist):
    cid = lax.axis_index("c")
    sid = lax.axis_index("s")
    wid = sid * NC + cid
    pltpu.sync_copy(dst3.at[wid], didx)

    def zbody(i, c):
        hist[pl.ds(pl.multiple_of(i * 16, 16), 16)] = jnp.zeros((16,), jnp.float32)
        return c

    lax.fori_loop(0, NPAD // 16, zbody, 0)
    ones = jnp.ones((16,), jnp.float32)

    def body(i, c):
        for j in range(CHUNK // 16):
            dvec = didx[i, pl.ds(j * 16, 16)]
            plsc.addupdate_scatter(hist, [dvec], ones)
        return c

    lax.fori_loop(0, NCH, body, 0)
    pltpu.sync_copy(hist, out.at[pl.ds(wid * NPAD, NPAD)])


def _build_deg_kernel(interpret=False):
    return pl.kernel(
        _deg_body,
        mesh=_mesh,
        out_type=jax.ShapeDtypeStruct((NW * NPAD,), jnp.float32),
        scratch_types=[
            pltpu.VMEM((NCH, CHUNK), jnp.int32),
            pltpu.VMEM((NPAD,), jnp.float32),
        ],
        compiler_params=pltpu.CompilerParams(needs_layout_passes=False),
        interpret=interpret,
    )


_deg_kernel = _build_deg_kernel()


def _edge_body(t, src3, dst3, zrows, out, sidx, didx, rows, agg_sh, sem):
    cid = lax.axis_index("c")
    sid = lax.axis_index("s")
    wid = sid * NC + cid
    pltpu.sync_copy(src3.at[wid], sidx)
    pltpu.sync_copy(dst3.at[wid], didx)
    pltpu.sync_copy(zrows, agg_sh.at[pl.ds(sid * STRIPE, STRIPE)])
    plsc.subcore_barrier()

    def body(i, carry):
        pltpu.async_copy(t.at[sidx.at[i]], rows, sem).wait()
        pltpu.sync_copy(rows, agg_sh.at[didx.at[i]], add=True)
        return carry

    lax.fori_loop(0, NCH, body, 0)
    plsc.subcore_barrier()
    pltpu.sync_copy(
        agg_sh.at[pl.ds(sid * STRIPE, STRIPE)],
        out.at[cid, pl.ds(sid * STRIPE, STRIPE)],
    )


def _build_edge_kernel(interpret=False):
    return pl.kernel(
        _edge_body,
        mesh=_mesh,
        out_type=jax.ShapeDtypeStruct((NC, NPAD, D), jnp.float32),
        scratch_types=[
            pltpu.VMEM((NCH, CHUNK), jnp.int32),
            pltpu.VMEM((NCH, CHUNK), jnp.int32),
            pltpu.VMEM((CHUNK, D), jnp.float32),
            pltpu.VMEM_SHARED((NPAD, D), jnp.float32),
            pltpu.SemaphoreType.DMA,
        ],
        interpret=interpret,
    )


_edge_kernel = _build_edge_kernel()


def _scale_from_degp(degp):
    ones_w = jnp.ones((NW, 1), jnp.float32)
    deg = lax.dot_general(
        degp, ones_w, (((0,), (0,)), ((), ())),
        preferred_element_type=jnp.float32,
        precision=lax.Precision.HIGHEST,
    )
    return lax.rsqrt(jnp.maximum(deg, 1.0))


def _tc_in_body(degp_ref, x_ref, u_ref):
    u_ref[...] = x_ref[...] * _scale_from_degp(degp_ref[...])


def _gcn_bn_relu(aggp, degp, w, b, g, be):
    s = _scale_from_degp(degp)
    agg = (aggp[0] + aggp[1]) * s
    y = jnp.dot(agg, w, preferred_element_type=jnp.float32) + b
    mask = lax.broadcasted_iota(jnp.int32, (NPAD, D), 0) < N
    mu = jnp.sum(jnp.where(mask, y, 0.0), axis=0, keepdims=True) / N
    dev = y - mu
    var = jnp.sum(jnp.where(mask, dev * dev, 0.0), axis=0, keepdims=True) / N
    h = dev * lax.rsqrt(var + 1e-5) * g + be
    return jnp.maximum(h, 0.0)


def _tc_mid_body(aggp_ref, degp_ref, b_ref, g_ref, be_ref, w_ref, u_ref):
    h = _gcn_bn_relu(aggp_ref[...], degp_ref[...], w_ref[...],
                     b_ref[...], g_ref[...], be_ref[...])
    u_ref[...] = h * _scale_from_degp(degp_ref[...])


def _tc_out_body(aggp_ref, degp_ref, b_ref, g_ref, be_ref, w_ref, out_ref):
    out_ref[...] = _gcn_bn_relu(aggp_ref[...], degp_ref[...], w_ref[...],
                                b_ref[...], g_ref[...], be_ref[...])


_tc_in = pl.pallas_call(
    _tc_in_body, out_shape=jax.ShapeDtypeStruct((NPAD, D), jnp.float32)
)
_tc_mid = pl.pallas_call(
    _tc_mid_body, out_shape=jax.ShapeDtypeStruct((NPAD, D), jnp.float32)
)
_tc_out = pl.pallas_call(
    _tc_out_body, out_shape=jax.ShapeDtypeStruct((NPAD, D), jnp.float32)
)


def kernel(x, edge_index, W1, b1, g1, be1, W2, b2, g2, be2):
    src = edge_index[0].astype(jnp.int32)
    dst = edge_index[1].astype(jnp.int32)
    pad_idx = jnp.full((EPAD - E,), N, dtype=jnp.int32)
    src3 = jnp.concatenate([src, pad_idx]).reshape(NW, NCH, CHUNK)
    dst3 = jnp.concatenate([dst, pad_idx]).reshape(NW, NCH, CHUNK)
    x_pad = jnp.concatenate([x, jnp.zeros((NPAD - N, D), jnp.float32)])

    zrows = jnp.zeros((STRIPE, D), jnp.float32)
    b1r, g1r, be1r = b1.reshape(1, D), g1.reshape(1, D), be1.reshape(1, D)
    b2r, g2r, be2r = b2.reshape(1, D), g2.reshape(1, D), be2.reshape(1, D)

    degp = _deg_kernel(dst3).reshape(NW, NPAD)
    u1 = _tc_in(degp, x_pad)
    aggp1 = _edge_kernel(u1, src3, dst3, zrows)
    u2 = _tc_mid(aggp1, degp, b1r, g1r, be1r, W1)
    aggp2 = _edge_kernel(u2, src3, dst3, zrows)
    out = _tc_out(aggp2, degp, b2r, g2r, be2r, W2)
    return out[:N]


# speedup vs baseline: 2.8231x; 2.8231x over previous
import functools

import jax
import jax.numpy as jnp
from jax import lax
from jax.experimental import pallas as pl
from jax.experimental.pallas import tpu as pltpu
from jax.experimental.pallas import tpu_sc as plsc

N = 10000
D = 128
E = 320000
NC = 2
NS = 16
NW = NC * NS
CHUNK = 128
NCH = 80
EPW = NCH * CHUNK
EPAD = NW * EPW
NPAD = 10112
STRIPE = NPAD // NS

_mesh = plsc.VectorSubcoreMesh(core_axis_name="c", subcore_axis_name="s")


def _deg_body(dst3, out, didx, hist):
    cid = lax.axis_index("c")
    sid = lax.axis_index("s")
    wid = sid * NC + cid
    pltpu.sync_copy(dst3.at[wid], didx)

    def zbody(i, c):
        hist[pl.ds(pl.multiple_of(i * 16, 16), 16)] = jnp.zeros((16,), jnp.float32)
        return c

    lax.fori_loop(0, NPAD // 16, zbody, 0)
    ones = jnp.ones((16,), jnp.float32)

    def body(i, c):
        for j in range(CHUNK // 16):
            dvec = didx[i, pl.ds(j * 16, 16)]
            plsc.addupdate_scatter(hist, [dvec], ones)
        return c

    lax.fori_loop(0, NCH, body, 0)
    pltpu.sync_copy(hist, out.at[pl.ds(wid * NPAD, NPAD)])


def _build_deg_kernel(interpret=False):
    return pl.kernel(
        _deg_body,
        mesh=_mesh,
        out_type=jax.ShapeDtypeStruct((NW * NPAD,), jnp.float32),
        scratch_types=[
            pltpu.VMEM((NCH, CHUNK), jnp.int32),
            pltpu.VMEM((NPAD,), jnp.float32),
        ],
        compiler_params=pltpu.CompilerParams(needs_layout_passes=False),
        interpret=interpret,
    )


_deg_kernel = _build_deg_kernel()


def _edge_body(t, src3, dst3, zrows, out, sidx, didx, rows, agg_sh, sem):
    cid = lax.axis_index("c")
    sid = lax.axis_index("s")
    wid = sid * NC + cid
    pltpu.sync_copy(src3.at[wid], sidx)
    pltpu.sync_copy(dst3.at[wid], didx)
    pltpu.sync_copy(zrows, agg_sh.at[pl.ds(sid * STRIPE, STRIPE)])
    plsc.subcore_barrier()

    def body(i, carry):
        pltpu.async_copy(t.at[sidx.at[i]], rows, sem).wait()
        pltpu.sync_copy(rows, agg_sh.at[didx.at[i]], add=True)
        return carry

    lax.fori_loop(0, NCH, body, 0)
    plsc.subcore_barrier()
    pltpu.sync_copy(
        agg_sh.at[pl.ds(sid * STRIPE, STRIPE)],
        out.at[cid, pl.ds(sid * STRIPE, STRIPE)],
    )


def _build_edge_kernel(interpret=False):
    return pl.kernel(
        _edge_body,
        mesh=_mesh,
        out_type=jax.ShapeDtypeStruct((NC, NPAD, D), jnp.float32),
        scratch_types=[
            pltpu.VMEM((NCH, CHUNK), jnp.int32),
            pltpu.VMEM((NCH, CHUNK), jnp.int32),
            pltpu.VMEM((CHUNK, D), jnp.float32),
            pltpu.VMEM_SHARED((NPAD, D), jnp.float32),
            pltpu.SemaphoreType.DMA,
        ],
        interpret=interpret,
    )


_edge_kernel = _build_edge_kernel()


def _scale_from_degp(degp):
    ones_w = jnp.ones((NW, 1), jnp.float32)
    deg = lax.dot_general(
        degp, ones_w, (((0,), (0,)), ((), ())),
        preferred_element_type=jnp.float32,
        precision=lax.Precision.HIGHEST,
    )
    return lax.rsqrt(jnp.maximum(deg, 1.0))


def _tc_in_body(degp_ref, x_ref, u_ref):
    u_ref[...] = x_ref[...] * _scale_from_degp(degp_ref[...])


def _gcn_bn_relu(aggp, degp, w, b, g, be):
    s = _scale_from_degp(degp)
    agg = (aggp[0] + aggp[1]) * s
    y = jnp.dot(agg, w, preferred_element_type=jnp.float32) + b
    mask = lax.broadcasted_iota(jnp.int32, (NPAD, D), 0) < N
    mu = jnp.sum(jnp.where(mask, y, 0.0), axis=0, keepdims=True) / N
    dev = y - mu
    var = jnp.sum(jnp.where(mask, dev * dev, 0.0), axis=0, keepdims=True) / N
    h = dev * lax.rsqrt(var + 1e-5) * g + be
    return jnp.maximum(h, 0.0)


def _tc_mid_body(aggp_ref, degp_ref, b_ref, g_ref, be_ref, w_ref, u_ref):
    h = _gcn_bn_relu(aggp_ref[...], degp_ref[...], w_ref[...],
                     b_ref[...], g_ref[...], be_ref[...])
    u_ref[...] = h * _scale_from_degp(degp_ref[...])


def _tc_out_body(aggp_ref, degp_ref, b_ref, g_ref, be_ref, w_ref, out_ref):
    out_ref[...] = _gcn_bn_relu(aggp_ref[...], degp_ref[...], w_ref[...],
                                b_ref[...], g_ref[...], be_ref[...])


_tc_in = pl.pallas_call(
    _tc_in_body, out_shape=jax.ShapeDtypeStruct((NPAD, D), jnp.float32)
)
_tc_mid = pl.pallas_call(
    _tc_mid_body, out_shape=jax.ShapeDtypeStruct((NPAD, D), jnp.float32)
)
_tc_out = pl.pallas_call(
    _tc_out_body, out_shape=jax.ShapeDtypeStruct((NPAD, D), jnp.float32)
)


def kernel(x, edge_index, W1, b1, g1, be1, W2, b2, g2, be2):
    src = edge_index[0].astype(jnp.int32)
    dst = edge_index[1].astype(jnp.int32)
    pad_idx = N + (jnp.arange(EPAD - E, dtype=jnp.int32) % (NPAD - N))
    src3 = jnp.concatenate([src, pad_idx]).reshape(NW, NCH, CHUNK)
    dst3 = jnp.concatenate([dst, pad_idx]).reshape(NW, NCH, CHUNK)
    x_pad = jnp.concatenate([x, jnp.zeros((NPAD - N, D), jnp.float32)])

    zrows = jnp.zeros((STRIPE, D), jnp.float32)
    b1r, g1r, be1r = b1.reshape(1, D), g1.reshape(1, D), be1.reshape(1, D)
    b2r, g2r, be2r = b2.reshape(1, D), g2.reshape(1, D), be2.reshape(1, D)

    degp = _deg_kernel(dst3).reshape(NW, NPAD)
    u1 = _tc_in(degp, x_pad)
    aggp1 = _edge_kernel(u1, src3, dst3, zrows)
    u2 = _tc_mid(aggp1, degp, b1r, g1r, be1r, W1)
    aggp2 = _edge_kernel(u2, src3, dst3, zrows)
    out = _tc_out(aggp2, degp, b2r, g2r, be2r, W2)
    return out[:N]


# speedup vs baseline: 3.5630x; 1.2621x over previous
import functools

import jax
import jax.numpy as jnp
from jax import lax
from jax.experimental import pallas as pl
from jax.experimental.pallas import tpu as pltpu
from jax.experimental.pallas import tpu_sc as plsc

N = 10000
D = 128
E = 320000
NC = 2
NS = 16
NW = NC * NS
CHUNK = 128
NCH = 80
EPW = NCH * CHUNK
EPAD = NW * EPW
NPAD = 10112
STRIPE = NPAD // NS

_mesh = plsc.VectorSubcoreMesh(core_axis_name="c", subcore_axis_name="s")


def _deg_body(dst3, out, didx, hist):
    cid = lax.axis_index("c")
    sid = lax.axis_index("s")
    wid = sid * NC + cid
    pltpu.sync_copy(dst3.at[wid], didx)

    def zbody(i, c):
        hist[pl.ds(pl.multiple_of(i * 16, 16), 16)] = jnp.zeros((16,), jnp.float32)
        return c

    lax.fori_loop(0, NPAD // 16, zbody, 0)
    ones = jnp.ones((16,), jnp.float32)

    def body(i, c):
        for j in range(CHUNK // 16):
            dvec = didx[i, pl.ds(j * 16, 16)]
            plsc.addupdate_scatter(hist, [dvec], ones)
        return c

    lax.fori_loop(0, NCH, body, 0)
    pltpu.sync_copy(hist, out.at[pl.ds(wid * NPAD, NPAD)])


def _build_deg_kernel(interpret=False):
    return pl.kernel(
        _deg_body,
        mesh=_mesh,
        out_type=jax.ShapeDtypeStruct((NW * NPAD,), jnp.float32),
        scratch_types=[
            pltpu.VMEM((NCH, CHUNK), jnp.int32),
            pltpu.VMEM((NPAD,), jnp.float32),
        ],
        compiler_params=pltpu.CompilerParams(needs_layout_passes=False),
        interpret=interpret,
    )


_deg_kernel = _build_deg_kernel()


def _edge_body(t, packed3, zrows, out, pk, sb0, sb1, sb2, sb3,
               db0, db1, db2, db3, rows0, rows1, agg_sh, g0, g1):
    cid = lax.axis_index("c")
    sid = lax.axis_index("s")
    wid = sid * NC + cid
    sb = [sb0, sb1, sb2, sb3]
    db = [db0, db1, db2, db3]
    rows = [rows0, rows1]
    gsem = [g0, g1]
    pltpu.sync_copy(packed3.at[wid], pk)
    pltpu.sync_copy(zrows, agg_sh.at[pl.ds(sid * STRIPE, STRIPE)])

    def unpack(c, slot):
        for j in range(CHUNK // 16):
            v = pk[c, pl.ds(j * 16, 16)]
            sb[slot][pl.ds(j * 16, 16)] = v & 0xFFFF
            db[slot][pl.ds(j * 16, 16)] = lax.shift_right_logical(v, 16)

    plsc.subcore_barrier()
    unpack(0, 0)
    unpack(1, 1)
    pltpu.async_copy(t.at[sb[0]], rows[0], gsem[0])

    def step(c, k):
        r = k % 2
        pltpu.make_async_copy(t.at[sb[k]], rows[r], gsem[r]).wait()

        @pl.when(c + 1 < NCH)
        def _():
            pltpu.async_copy(t.at[sb[(k + 1) % 4]], rows[1 - r], gsem[1 - r])

        pltpu.sync_copy(rows[r], agg_sh.at[db[k]], add=True)
        unpack(jnp.minimum(c + 2, NCH - 1), (k + 2) % 4)

    def body(i, carry):
        c0 = 4 * i
        for k in range(4):
            step(c0 + k, k)
        return carry

    lax.fori_loop(0, NCH // 4, body, 0)
    plsc.subcore_barrier()
    pltpu.sync_copy(
        agg_sh.at[pl.ds(sid * STRIPE, STRIPE)],
        out.at[cid, pl.ds(sid * STRIPE, STRIPE)],
    )


def _build_edge_kernel(interpret=False):
    return pl.kernel(
        _edge_body,
        mesh=_mesh,
        out_type=jax.ShapeDtypeStruct((NC, NPAD, D), jnp.float32),
        scratch_types=[
            pltpu.VMEM((NCH, CHUNK), jnp.int32),
            pltpu.VMEM((CHUNK,), jnp.int32),
            pltpu.VMEM((CHUNK,), jnp.int32),
            pltpu.VMEM((CHUNK,), jnp.int32),
            pltpu.VMEM((CHUNK,), jnp.int32),
            pltpu.VMEM((CHUNK,), jnp.int32),
            pltpu.VMEM((CHUNK,), jnp.int32),
            pltpu.VMEM((CHUNK,), jnp.int32),
            pltpu.VMEM((CHUNK,), jnp.int32),
            pltpu.VMEM((CHUNK, D), jnp.float32),
            pltpu.VMEM((CHUNK, D), jnp.float32),
            pltpu.VMEM_SHARED((NPAD, D), jnp.float32),
            pltpu.SemaphoreType.DMA,
            pltpu.SemaphoreType.DMA,
        ],
        compiler_params=pltpu.CompilerParams(needs_layout_passes=False),
        interpret=interpret,
    )


_edge_kernel = _build_edge_kernel()


def _scale_from_degp(degp):
    ones_w = jnp.ones((NW, 1), jnp.float32)
    deg = lax.dot_general(
        degp, ones_w, (((0,), (0,)), ((), ())),
        preferred_element_type=jnp.float32,
        precision=lax.Precision.HIGHEST,
    )
    return lax.rsqrt(jnp.maximum(deg, 1.0))


def _tc_in_body(degp_ref, x_ref, u_ref):
    u_ref[...] = x_ref[...] * _scale_from_degp(degp_ref[...])


def _gcn_bn_relu(aggp, degp, w, b, g, be):
    s = _scale_from_degp(degp)
    agg = (aggp[0] + aggp[1]) * s
    y = jnp.dot(agg, w, preferred_element_type=jnp.float32) + b
    mask = lax.broadcasted_iota(jnp.int32, (NPAD, D), 0) < N
    mu = jnp.sum(jnp.where(mask, y, 0.0), axis=0, keepdims=True) / N
    dev = y - mu
    var = jnp.sum(jnp.where(mask, dev * dev, 0.0), axis=0, keepdims=True) / N
    h = dev * lax.rsqrt(var + 1e-5) * g + be
    return jnp.maximum(h, 0.0)


def _tc_mid_body(aggp_ref, degp_ref, b_ref, g_ref, be_ref, w_ref, u_ref):
    h = _gcn_bn_relu(aggp_ref[...], degp_ref[...], w_ref[...],
                     b_ref[...], g_ref[...], be_ref[...])
    u_ref[...] = h * _scale_from_degp(degp_ref[...])


def _tc_out_body(aggp_ref, degp_ref, b_ref, g_ref, be_ref, w_ref, out_ref):
    out_ref[...] = _gcn_bn_relu(aggp_ref[...], degp_ref[...], w_ref[...],
                                b_ref[...], g_ref[...], be_ref[...])


_tc_in = pl.pallas_call(
    _tc_in_body, out_shape=jax.ShapeDtypeStruct((NPAD, D), jnp.float32)
)
_tc_mid = pl.pallas_call(
    _tc_mid_body, out_shape=jax.ShapeDtypeStruct((NPAD, D), jnp.float32)
)
_tc_out = pl.pallas_call(
    _tc_out_body, out_shape=jax.ShapeDtypeStruct((NPAD, D), jnp.float32)
)


def kernel(x, edge_index, W1, b1, g1, be1, W2, b2, g2, be2):
    src = edge_index[0].astype(jnp.int32)
    dst = edge_index[1].astype(jnp.int32)
    pad_idx = N + (jnp.arange(EPAD - E, dtype=jnp.int32) % (NPAD - N))
    src3 = jnp.concatenate([src, pad_idx]).reshape(NW, NCH, CHUNK)
    dst3 = jnp.concatenate([dst, pad_idx]).reshape(NW, NCH, CHUNK)
    packed3 = src3 | (dst3 << 16)
    x_pad = jnp.concatenate([x, jnp.zeros((NPAD - N, D), jnp.float32)])

    zrows = jnp.zeros((STRIPE, D), jnp.float32)
    b1r, g1r, be1r = b1.reshape(1, D), g1.reshape(1, D), be1.reshape(1, D)
    b2r, g2r, be2r = b2.reshape(1, D), g2.reshape(1, D), be2.reshape(1, D)

    degp = _deg_kernel(dst3).reshape(NW, NPAD)
    u1 = _tc_in(degp, x_pad)
    aggp1 = _edge_kernel(u1, packed3, zrows)
    u2 = _tc_mid(aggp1, degp, b1r, g1r, be1r, W1)
    aggp2 = _edge_kernel(u2, packed3, zrows)
    out = _tc_out(aggp2, degp, b2r, g2r, be2r, W2)
    return out[:N]


# speedup vs baseline: 3.5649x; 1.0005x over previous
import functools

import jax
import jax.numpy as jnp
from jax import lax
from jax.experimental import pallas as pl
from jax.experimental.pallas import tpu as pltpu
from jax.experimental.pallas import tpu_sc as plsc

N = 10000
D = 128
E = 320000
NC = 2
NS = 16
NW = NC * NS
CHUNK = 128
NCH = 80
EPW = NCH * CHUNK
EPAD = NW * EPW
NPAD = 10112
STRIPE = NPAD // NS

_mesh = plsc.VectorSubcoreMesh(core_axis_name="c", subcore_axis_name="s")


def _deg_body(dst3, out, didx, hist):
    cid = lax.axis_index("c")
    sid = lax.axis_index("s")
    wid = sid * NC + cid
    pltpu.sync_copy(dst3.at[wid], didx)

    def zbody(i, c):
        hist[pl.ds(pl.multiple_of(i * 16, 16), 16)] = jnp.zeros((16,), jnp.float32)
        return c

    lax.fori_loop(0, NPAD // 16, zbody, 0)
    ones = jnp.ones((16,), jnp.float32)

    def body(i, c):
        for j in range(CHUNK // 16):
            dvec = didx[i, pl.ds(j * 16, 16)]
            plsc.addupdate_scatter(hist, [dvec], ones)
        return c

    lax.fori_loop(0, NCH, body, 0)
    pltpu.sync_copy(hist, out.at[pl.ds(wid * NPAD, NPAD)])


def _build_deg_kernel(interpret=False):
    return pl.kernel(
        _deg_body,
        mesh=_mesh,
        out_type=jax.ShapeDtypeStruct((NW * NPAD,), jnp.float32),
        scratch_types=[
            pltpu.VMEM((NCH, CHUNK), jnp.int32),
            pltpu.VMEM((NPAD,), jnp.float32),
        ],
        compiler_params=pltpu.CompilerParams(needs_layout_passes=False),
        interpret=interpret,
    )


_deg_kernel = _build_deg_kernel()


def _edge_body(t, packed3, zrows, out, pk, sb0, sb1, sb2, sb3,
               db0, db1, db2, db3, rows0, rows1, agg_sh, g0, g1, s0, s1):
    cid = lax.axis_index("c")
    sid = lax.axis_index("s")
    wid = sid * NC + cid
    sb = [sb0, sb1, sb2, sb3]
    db = [db0, db1, db2, db3]
    rows = [rows0, rows1]
    gsem = [g0, g1]
    ssem = [s0, s1]
    pltpu.sync_copy(packed3.at[wid], pk)
    pltpu.sync_copy(zrows, agg_sh.at[pl.ds(sid * STRIPE, STRIPE)])

    def unpack(c, slot):
        for j in range(CHUNK // 16):
            v = pk[c, pl.ds(j * 16, 16)]
            sb[slot][pl.ds(j * 16, 16)] = v & 0xFFFF
            db[slot][pl.ds(j * 16, 16)] = lax.shift_right_logical(v, 16)

    plsc.subcore_barrier()
    unpack(0, 0)
    unpack(1, 1)
    pltpu.async_copy(t.at[sb[0]], rows[0], gsem[0])

    def step(c, k):
        r = k % 2
        pltpu.make_async_copy(t.at[sb[k]], rows[r], gsem[r]).wait()

        @pl.when(c >= 1)
        def _():
            pltpu.make_async_copy(rows[1 - r], agg_sh.at[db[(k + 3) % 4]],
                                  ssem[1 - r]).wait()

        @pl.when(c + 1 < NCH)
        def _():
            pltpu.async_copy(t.at[sb[(k + 1) % 4]], rows[1 - r], gsem[1 - r])

        pltpu.async_copy(rows[r], agg_sh.at[db[k]], ssem[r], add=True)
        unpack(jnp.minimum(c + 2, NCH - 1), (k + 2) % 4)

    def body(i, carry):
        c0 = 4 * i
        for k in range(4):
            step(c0 + k, k)
        return carry

    lax.fori_loop(0, NCH // 4, body, 0)
    pltpu.make_async_copy(rows[1], agg_sh.at[db[3]], ssem[1]).wait()
    plsc.subcore_barrier()
    pltpu.sync_copy(
        agg_sh.at[pl.ds(sid * STRIPE, STRIPE)],
        out.at[cid, pl.ds(sid * STRIPE, STRIPE)],
    )


def _build_edge_kernel(interpret=False):
    return pl.kernel(
        _edge_body,
        mesh=_mesh,
        out_type=jax.ShapeDtypeStruct((NC, NPAD, D), jnp.float32),
        scratch_types=[
            pltpu.VMEM((NCH, CHUNK), jnp.int32),
            pltpu.VMEM((CHUNK,), jnp.int32),
            pltpu.VMEM((CHUNK,), jnp.int32),
            pltpu.VMEM((CHUNK,), jnp.int32),
            pltpu.VMEM((CHUNK,), jnp.int32),
            pltpu.VMEM((CHUNK,), jnp.int32),
            pltpu.VMEM((CHUNK,), jnp.int32),
            pltpu.VMEM((CHUNK,), jnp.int32),
            pltpu.VMEM((CHUNK,), jnp.int32),
            pltpu.VMEM((CHUNK, D), jnp.float32),
            pltpu.VMEM((CHUNK, D), jnp.float32),
            pltpu.VMEM_SHARED((NPAD, D), jnp.float32),
            pltpu.SemaphoreType.DMA,
            pltpu.SemaphoreType.DMA,
            pltpu.SemaphoreType.DMA,
            pltpu.SemaphoreType.DMA,
        ],
        compiler_params=pltpu.CompilerParams(needs_layout_passes=False),
        interpret=interpret,
    )


_edge_kernel = _build_edge_kernel()


def _scale_from_degp(degp):
    ones_w = jnp.ones((NW, 1), jnp.float32)
    deg = lax.dot_general(
        degp, ones_w, (((0,), (0,)), ((), ())),
        preferred_element_type=jnp.float32,
        precision=lax.Precision.HIGHEST,
    )
    return lax.rsqrt(jnp.maximum(deg, 1.0))


def _tc_in_body(degp_ref, x_ref, u_ref):
    u_ref[...] = x_ref[...] * _scale_from_degp(degp_ref[...])


def _gcn_bn_relu(aggp, degp, w, b, g, be):
    s = _scale_from_degp(degp)
    agg = (aggp[0] + aggp[1]) * s
    y = jnp.dot(agg, w, preferred_element_type=jnp.float32) + b
    mask = lax.broadcasted_iota(jnp.int32, (NPAD, D), 0) < N
    mu = jnp.sum(jnp.where(mask, y, 0.0), axis=0, keepdims=True) / N
    dev = y - mu
    var = jnp.sum(jnp.where(mask, dev * dev, 0.0), axis=0, keepdims=True) / N
    h = dev * lax.rsqrt(var + 1e-5) * g + be
    return jnp.maximum(h, 0.0)


def _tc_mid_body(aggp_ref, degp_ref, b_ref, g_ref, be_ref, w_ref, u_ref):
    h = _gcn_bn_relu(aggp_ref[...], degp_ref[...], w_ref[...],
                     b_ref[...], g_ref[...], be_ref[...])
    u_ref[...] = h * _scale_from_degp(degp_ref[...])


def _tc_out_body(aggp_ref, degp_ref, b_ref, g_ref, be_ref, w_ref, out_ref):
    out_ref[...] = _gcn_bn_relu(aggp_ref[...], degp_ref[...], w_ref[...],
                                b_ref[...], g_ref[...], be_ref[...])


_tc_in = pl.pallas_call(
    _tc_in_body, out_shape=jax.ShapeDtypeStruct((NPAD, D), jnp.float32)
)
_tc_mid = pl.pallas_call(
    _tc_mid_body, out_shape=jax.ShapeDtypeStruct((NPAD, D), jnp.float32)
)
_tc_out = pl.pallas_call(
    _tc_out_body, out_shape=jax.ShapeDtypeStruct((NPAD, D), jnp.float32)
)


def kernel(x, edge_index, W1, b1, g1, be1, W2, b2, g2, be2):
    src = edge_index[0].astype(jnp.int32)
    dst = edge_index[1].astype(jnp.int32)
    pad_idx = N + (jnp.arange(EPAD - E, dtype=jnp.int32) % (NPAD - N))
    src3 = jnp.concatenate([src, pad_idx]).reshape(NW, NCH, CHUNK)
    dst3 = jnp.concatenate([dst, pad_idx]).reshape(NW, NCH, CHUNK)
    packed3 = src3 | (dst3 << 16)
    x_pad = jnp.concatenate([x, jnp.zeros((NPAD - N, D), jnp.float32)])

    zrows = jnp.zeros((STRIPE, D), jnp.float32)
    b1r, g1r, be1r = b1.reshape(1, D), g1.reshape(1, D), be1.reshape(1, D)
    b2r, g2r, be2r = b2.reshape(1, D), g2.reshape(1, D), be2.reshape(1, D)

    degp = _deg_kernel(dst3).reshape(NW, NPAD)
    u1 = _tc_in(degp, x_pad)
    aggp1 = _edge_kernel(u1, packed3, zrows)
    u2 = _tc_mid(aggp1, degp, b1r, g1r, be1r, W1)
    aggp2 = _edge_kernel(u2, packed3, zrows)
    out = _tc_out(aggp2, degp, b2r, g2r, be2r, W2)
    return out[:N]


# speedup vs baseline: 3.5688x; 1.0011x over previous
import jax
import jax.numpy as jnp
from jax import lax
from jax.experimental import pallas as pl
from jax.experimental.pallas import tpu as pltpu
from jax.experimental.pallas import tpu_sc as plsc

N = 10000
D = 128
E = 320000
NC = 2
NS = 16
NW = NC * NS
CHUNK = 128
NCH = 80
EPW = NCH * CHUNK
EPAD = NW * EPW
NPAD = 10112
STRIPE = NPAD // NS

_mesh = plsc.VectorSubcoreMesh(core_axis_name="c", subcore_axis_name="s")


def _deg_body(dst3, out, didx, hist):
    cid = lax.axis_index("c")
    sid = lax.axis_index("s")
    wid = sid * NC + cid
    pltpu.sync_copy(dst3.at[wid], didx)

    def zbody(i, c):
        hist[pl.ds(pl.multiple_of(i * 16, 16), 16)] = jnp.zeros((16,), jnp.float32)
        return c

    lax.fori_loop(0, NPAD // 16, zbody, 0)
    ones = jnp.ones((16,), jnp.float32)

    def body(i, c):
        for j in range(CHUNK // 16):
            dvec = didx[i, pl.ds(j * 16, 16)]
            plsc.addupdate_scatter(hist, [dvec], ones)
        return c

    lax.fori_loop(0, NCH, body, 0)
    pltpu.sync_copy(hist, out.at[pl.ds(wid * NPAD, NPAD)])


def _build_deg_kernel(interpret=False):
    return pl.kernel(
        _deg_body,
        mesh=_mesh,
        out_type=jax.ShapeDtypeStruct((NW * NPAD,), jnp.float32),
        scratch_types=[
            pltpu.VMEM((NCH, CHUNK), jnp.int32),
            pltpu.VMEM((NPAD,), jnp.float32),
        ],
        compiler_params=pltpu.CompilerParams(needs_layout_passes=False),
        interpret=interpret,
    )


_deg_kernel = _build_deg_kernel()


def _edge_body(t, packed3, zrows, out, pk, sb0, sb1, sb2, sb3,
               db0, db1, db2, db3, rows0, rows1, agg_sh, g0, g1, s0, s1):
    cid = lax.axis_index("c")
    sid = lax.axis_index("s")
    wid = sid * NC + cid
    sb = [sb0, sb1, sb2, sb3]
    db = [db0, db1, db2, db3]
    rows = [rows0, rows1]
    gsem = [g0, g1]
    ssem = [s0, s1]
    pltpu.sync_copy(packed3.at[wid], pk)
    pltpu.sync_copy(zrows, agg_sh.at[pl.ds(sid * STRIPE, STRIPE)])

    def unpack(c, slot):
        for j in range(CHUNK // 16):
            v = pk[c, pl.ds(j * 16, 16)]
            sb[slot][pl.ds(j * 16, 16)] = v & 0xFFFF
            db[slot][pl.ds(j * 16, 16)] = lax.shift_right_logical(v, 16)

    plsc.subcore_barrier()
    unpack(0, 0)
    unpack(1, 1)
    pltpu.async_copy(t.at[sb[0]], rows[0], gsem[0])

    def step(c, k):
        r = k % 2
        pltpu.make_async_copy(t.at[sb[k]], rows[r], gsem[r]).wait()

        @pl.when(c >= 1)
        def _():
            pltpu.make_async_copy(rows[1 - r], agg_sh.at[db[(k + 3) % 4]],
                                  ssem[1 - r]).wait()

        @pl.when(c + 1 < NCH)
        def _():
            pltpu.async_copy(t.at[sb[(k + 1) % 4]], rows[1 - r], gsem[1 - r])

        pltpu.async_copy(rows[r], agg_sh.at[db[k]], ssem[r], add=True)
        unpack(jnp.minimum(c + 2, NCH - 1), (k + 2) % 4)

    def body(i, carry):
        c0 = 4 * i
        for k in range(4):
            step(c0 + k, k)
        return carry

    lax.fori_loop(0, NCH // 4, body, 0)
    pltpu.make_async_copy(rows[1], agg_sh.at[db[3]], ssem[1]).wait()
    plsc.subcore_barrier()
    pltpu.sync_copy(
        agg_sh.at[pl.ds(sid * STRIPE, STRIPE)],
        out.at[cid, pl.ds(sid * STRIPE, STRIPE)],
    )


def _build_edge_kernel(interpret=False):
    return pl.kernel(
        _edge_body,
        mesh=_mesh,
        out_type=jax.ShapeDtypeStruct((NC, NPAD, D), jnp.float32),
        scratch_types=[
            pltpu.VMEM((NCH, CHUNK), jnp.int32),
            pltpu.VMEM((CHUNK,), jnp.int32),
            pltpu.VMEM((CHUNK,), jnp.int32),
            pltpu.VMEM((CHUNK,), jnp.int32),
            pltpu.VMEM((CHUNK,), jnp.int32),
            pltpu.VMEM((CHUNK,), jnp.int32),
            pltpu.VMEM((CHUNK,), jnp.int32),
            pltpu.VMEM((CHUNK,), jnp.int32),
            pltpu.VMEM((CHUNK,), jnp.int32),
            pltpu.VMEM((CHUNK, D), jnp.float32),
            pltpu.VMEM((CHUNK, D), jnp.float32),
            pltpu.VMEM_SHARED((NPAD, D), jnp.float32),
            pltpu.SemaphoreType.DMA,
            pltpu.SemaphoreType.DMA,
            pltpu.SemaphoreType.DMA,
            pltpu.SemaphoreType.DMA,
        ],
        compiler_params=pltpu.CompilerParams(needs_layout_passes=False),
        interpret=interpret,
    )


_edge_kernel = _build_edge_kernel()


def _scale_from_degp(degp):
    ones_w = jnp.ones((NW, 1), jnp.float32)
    deg = lax.dot_general(
        degp, ones_w, (((0,), (0,)), ((), ())),
        preferred_element_type=jnp.float32,
        precision=lax.Precision.HIGHEST,
    )
    return lax.rsqrt(jnp.maximum(deg, 1.0))


def _tc_in_body(degp_ref, x_ref, u_ref):
    u_ref[...] = x_ref[...] * _scale_from_degp(degp_ref[...])


def _gcn_bn_relu(aggp, degp, w, b, g, be):
    s = _scale_from_degp(degp)
    agg = (aggp[0] + aggp[1]) * s
    y = jnp.dot(agg, w, preferred_element_type=jnp.float32) + b
    mask = lax.broadcasted_iota(jnp.int32, (NPAD, D), 0) < N
    mu = jnp.sum(jnp.where(mask, y, 0.0), axis=0, keepdims=True) / N
    dev = y - mu
    var = jnp.sum(jnp.where(mask, dev * dev, 0.0), axis=0, keepdims=True) / N
    h = dev * lax.rsqrt(var + 1e-5) * g + be
    return jnp.maximum(h, 0.0)


def _tc_mid_body(aggp_ref, degp_ref, b_ref, g_ref, be_ref, w_ref, u_ref):
    h = _gcn_bn_relu(aggp_ref[...], degp_ref[...], w_ref[...],
                     b_ref[...], g_ref[...], be_ref[...])
    u_ref[...] = h * _scale_from_degp(degp_ref[...])


def _tc_out_body(aggp_ref, degp_ref, b_ref, g_ref, be_ref, w_ref, out_ref):
    out_ref[...] = _gcn_bn_relu(aggp_ref[...], degp_ref[...], w_ref[...],
                                b_ref[...], g_ref[...], be_ref[...])


_tc_in = pl.pallas_call(
    _tc_in_body, out_shape=jax.ShapeDtypeStruct((NPAD, D), jnp.float32)
)
_tc_mid = pl.pallas_call(
    _tc_mid_body, out_shape=jax.ShapeDtypeStruct((NPAD, D), jnp.float32)
)
_tc_out = pl.pallas_call(
    _tc_out_body, out_shape=jax.ShapeDtypeStruct((NPAD, D), jnp.float32)
)


def kernel(x, edge_index, W1, b1, g1, be1, W2, b2, g2, be2):
    src = edge_index[0].astype(jnp.int32)
    dst = edge_index[1].astype(jnp.int32)
    pad_idx = N + (jnp.arange(EPAD - E, dtype=jnp.int32) % (NPAD - N))
    src3 = jnp.concatenate([src, pad_idx]).reshape(NW, NCH, CHUNK)
    dst3 = jnp.concatenate([dst, pad_idx]).reshape(NW, NCH, CHUNK)
    packed3 = src3 | (dst3 << 16)
    x_pad = jnp.concatenate([x, jnp.zeros((NPAD - N, D), jnp.float32)])

    zrows = jnp.zeros((STRIPE, D), jnp.float32)
    b1r, g1r, be1r = b1.reshape(1, D), g1.reshape(1, D), be1.reshape(1, D)
    b2r, g2r, be2r = b2.reshape(1, D), g2.reshape(1, D), be2.reshape(1, D)

    degp = _deg_kernel(dst3).reshape(NW, NPAD)
    u1 = _tc_in(degp, x_pad)
    aggp1 = _edge_kernel(u1, packed3, zrows)
    u2 = _tc_mid(aggp1, degp, b1r, g1r, be1r, W1)
    aggp2 = _edge_kernel(u2, packed3, zrows)
    out = _tc_out(aggp2, degp, b2r, g2r, be2r, W2)
    return out[:N]
